# baseline (speedup 1.0000x reference)
import math

import jax
import jax.numpy as jnp
from jax.experimental import pallas as pl
from jax.experimental.pallas import tpu as pltpu

_TN = 512
_GM = 512


def _dot_nt(a, b):
    return jax.lax.dot_general(a, b, (((1,), (1,)), ((), ())),
                               preferred_element_type=jnp.float32)


def _dot_nn(a, b):
    return jax.lax.dot_general(a, b, (((1,), (0,)), ((), ())),
                               preferred_element_type=jnp.float32)


def _fused_body(wt_ref, x_ref, a_ref, dinv_ref, parts_ref):
    a = a_ref[...]
    ones = jnp.ones((1, a.shape[1]), jnp.float32)
    deg = _dot_nt(ones, a)
    dinv = jax.lax.rsqrt(jnp.maximum(deg, 1.0))
    dinv_ref[0] = dinv
    y = dinv * _dot_nt(wt_ref[...], x_ref[...])
    parts_ref[0] = _dot_nn(y, a)


def _combine_body(b_ref, parts_ref, dinv_ref, neg_ref, tanh_ref):
    nparts, _, n = parts_ref.shape
    num_graphs = neg_ref.shape[0]
    per = neg_ref.shape[2]
    acc = parts_ref[0]
    for p in range(1, nparts):
        acc = acc + parts_ref[p]
    s = acc * dinv_ref[0] + b_ref[0]
    for g in range(num_graphs):
        sg = s[:, g * per:(g + 1) * per]
        neg_ref[g] = -sg
        tanh_ref[g] = jnp.tanh(sg)


def _gather_body(perm_ref, ts_ref, feat_ref, out_ref):
    base = pl.program_id(0) * _GM
    rows = []
    for mi in range(_GM):
        r = perm_ref[base + mi]
        chunk = feat_ref[r >> 3]
        row = pltpu.roll(chunk, -(r & 7), axis=0)[0:1, :]
        rows.append(row * ts_ref[r])
    for mi in range(_GM):
        out_ref[mi] = rows[mi][0]


def kernel(adj, feature, weight, bias):
    n, f = feature.shape
    num_graphs = 8
    per_graph = n // num_graphs
    k = int(math.ceil(0.5 * per_graph))
    wt = weight.reshape(1, f)

    nstrip = n // _TN
    dinv, parts = pl.pallas_call(
        _fused_body,
        out_shape=(jax.ShapeDtypeStruct((1, 1, n), jnp.float32),
                   jax.ShapeDtypeStruct((nstrip, 1, n), jnp.float32)),
        grid=(nstrip,),
        in_specs=[
            pl.BlockSpec((1, f), lambda i: (0, 0)),
            pl.BlockSpec((_TN, f), lambda i: (i, 0)),
            pl.BlockSpec((_TN, n), lambda i: (i, 0)),
        ],
        out_specs=(
            pl.BlockSpec((1, 1, _TN), lambda i: (0, 0, i)),
            pl.BlockSpec((1, 1, n), lambda i: (i, 0, 0)),
        ),
        compiler_params=pltpu.CompilerParams(
            dimension_semantics=("parallel",),
            vmem_limit_bytes=48 * 1024 * 1024),
    )(wt, feature, adj)

    neg3, tanh3 = pl.pallas_call(
        _combine_body,
        out_shape=(jax.ShapeDtypeStruct((num_graphs, 1, per_graph),
                                        jnp.float32),
                   jax.ShapeDtypeStruct((num_graphs, 1, per_graph),
                                        jnp.float32)),
        in_specs=[
            pl.BlockSpec(memory_space=pltpu.MemorySpace.SMEM),
            pl.BlockSpec((nstrip, 1, n), lambda: (0, 0, 0)),
            pl.BlockSpec((1, 1, n), lambda: (0, 0, 0)),
        ],
        out_specs=(
            pl.BlockSpec((num_graphs, 1, per_graph), lambda: (0, 0, 0)),
            pl.BlockSpec((num_graphs, 1, per_graph), lambda: (0, 0, 0)),
        ),
    )(bias, parts, dinv)

    order = jnp.argsort(neg3.reshape(num_graphs, per_graph), axis=1)
    offs = (jnp.arange(num_graphs, dtype=jnp.int32) * per_graph)[:, None]
    perm = (order[:, :k].astype(jnp.int32) + offs).reshape(-1)

    pooled = pl.pallas_call(
        _gather_body,
        out_shape=jax.ShapeDtypeStruct((perm.shape[0], f), jnp.float32),
        grid_spec=pltpu.PrefetchScalarGridSpec(
            num_scalar_prefetch=2,
            grid=(perm.shape[0] // _GM,),
            in_specs=[
                pl.BlockSpec((n // 8, 8, f), lambda i, perm, ts: (0, 0, 0)),
            ],
            out_specs=pl.BlockSpec((_GM, f), lambda i, perm, ts: (i, 0)),
        ),
        compiler_params=pltpu.CompilerParams(
            dimension_semantics=("parallel",)),
    )(perm, tanh3.reshape(-1), feature.reshape(n // 8, 8, f))

    next_batch_num_nodes = jnp.full((num_graphs,), k, dtype=jnp.int32)
    return pooled, perm, next_batch_num_nodes


# speedup vs baseline: 1.0745x; 1.0745x over previous
import math

import jax
import jax.numpy as jnp
from jax.experimental import pallas as pl
from jax.experimental.pallas import tpu as pltpu

_TN = 512
_GM = 512


def _dot_nt(a, b):
    return jax.lax.dot_general(a, b, (((1,), (1,)), ((), ())),
                               preferred_element_type=jnp.float32)


def _dot_nn(a, b):
    return jax.lax.dot_general(a, b, (((1,), (0,)), ((), ())),
                               preferred_element_type=jnp.float32)


def _fused_body(wt_ref, x_ref, a_ref, dinv_ref, parts_ref):
    a = a_ref[...]
    ones = jnp.ones((1, a.shape[1]), jnp.float32)
    deg = _dot_nt(ones, a)
    dinv = jax.lax.rsqrt(jnp.maximum(deg, 1.0))
    dinv_ref[0] = dinv
    y = dinv * _dot_nt(wt_ref[...], x_ref[...])
    parts_ref[0] = _dot_nn(y, a)


def _combine_body(b_ref, parts_ref, dinv_ref, neg_ref, tanh_ref):
    nparts, _, n = parts_ref.shape
    num_graphs = neg_ref.shape[0]
    per = neg_ref.shape[2]
    acc = parts_ref[0]
    for p in range(1, nparts):
        acc = acc + parts_ref[p]
    s = acc * dinv_ref[0] + b_ref[0]
    for g in range(num_graphs):
        sg = s[:, g * per:(g + 1) * per]
        neg_ref[g] = -sg
        tanh_ref[g] = jnp.tanh(sg)


def _gather_body(perm_ref, ts_ref, feat_ref, out_ref):
    base = pl.program_id(0) * _GM
    rows = []
    for mi in range(_GM):
        r = perm_ref[base + mi]
        rows.append(feat_ref[r, 0] * ts_ref[r])
    for mi in range(_GM):
        out_ref[mi] = rows[mi]


def kernel(adj, feature, weight, bias):
    n, f = feature.shape
    num_graphs = 8
    per_graph = n // num_graphs
    k = int(math.ceil(0.5 * per_graph))
    wt = weight.reshape(1, f)

    nstrip = n // _TN
    dinv, parts = pl.pallas_call(
        _fused_body,
        out_shape=(jax.ShapeDtypeStruct((1, 1, n), jnp.float32),
                   jax.ShapeDtypeStruct((nstrip, 1, n), jnp.float32)),
        grid=(nstrip,),
        in_specs=[
            pl.BlockSpec((1, f), lambda i: (0, 0)),
            pl.BlockSpec((_TN, f), lambda i: (i, 0)),
            pl.BlockSpec((_TN, n), lambda i: (i, 0)),
        ],
        out_specs=(
            pl.BlockSpec((1, 1, _TN), lambda i: (0, 0, i)),
            pl.BlockSpec((1, 1, n), lambda i: (i, 0, 0)),
        ),
        compiler_params=pltpu.CompilerParams(
            dimension_semantics=("parallel",),
            vmem_limit_bytes=48 * 1024 * 1024),
    )(wt, feature, adj)

    neg3, tanh3 = pl.pallas_call(
        _combine_body,
        out_shape=(jax.ShapeDtypeStruct((num_graphs, 1, per_graph),
                                        jnp.float32),
                   jax.ShapeDtypeStruct((num_graphs, 1, per_graph),
                                        jnp.float32)),
        in_specs=[
            pl.BlockSpec(memory_space=pltpu.MemorySpace.SMEM),
            pl.BlockSpec((nstrip, 1, n), lambda: (0, 0, 0)),
            pl.BlockSpec((1, 1, n), lambda: (0, 0, 0)),
        ],
        out_specs=(
            pl.BlockSpec((num_graphs, 1, per_graph), lambda: (0, 0, 0)),
            pl.BlockSpec((num_graphs, 1, per_graph), lambda: (0, 0, 0)),
        ),
    )(bias, parts, dinv)

    order = jnp.argsort(neg3.reshape(num_graphs, per_graph), axis=1)
    offs = (jnp.arange(num_graphs, dtype=jnp.int32) * per_graph)[:, None]
    perm = (order[:, :k].astype(jnp.int32) + offs).reshape(-1)

    pooled = pl.pallas_call(
        _gather_body,
        out_shape=jax.ShapeDtypeStruct((perm.shape[0], f), jnp.float32),
        grid_spec=pltpu.PrefetchScalarGridSpec(
            num_scalar_prefetch=2,
            grid=(perm.shape[0] // _GM,),
            in_specs=[
                pl.BlockSpec((n, 1, f), lambda i, perm, ts: (0, 0, 0)),
            ],
            out_specs=pl.BlockSpec((_GM, f), lambda i, perm, ts: (i, 0)),
        ),
        compiler_params=pltpu.CompilerParams(
            dimension_semantics=("parallel",)),
    )(perm, tanh3.reshape(-1), feature.reshape(n, 1, f))

    next_batch_num_nodes = jnp.full((num_graphs,), k, dtype=jnp.int32)
    return pooled, perm, next_batch_num_nodes


# speedup vs baseline: 1.0812x; 1.0062x over previous
import math

import jax
import jax.numpy as jnp
from jax.experimental import pallas as pl
from jax.experimental.pallas import tpu as pltpu

_TN = 512
_GM = 1024


def _dot_nt(a, b):
    return jax.lax.dot_general(a, b, (((1,), (1,)), ((), ())),
                               preferred_element_type=jnp.float32)


def _dot_nn(a, b):
    return jax.lax.dot_general(a, b, (((1,), (0,)), ((), ())),
                               preferred_element_type=jnp.float32)


def _fused_body(wt_ref, x_ref, a_ref, dinv_ref, parts_ref):
    a = a_ref[...]
    ones = jnp.ones((1, a.shape[1]), jnp.float32)
    deg = _dot_nt(ones, a)
    dinv = jax.lax.rsqrt(jnp.maximum(deg, 1.0))
    dinv_ref[0] = dinv
    y = dinv * _dot_nt(wt_ref[...], x_ref[...])
    parts_ref[0] = _dot_nn(y, a)


def _combine_body(b_ref, parts_ref, dinv_ref, neg_ref, tanh_ref):
    nparts, _, n = parts_ref.shape
    num_graphs = neg_ref.shape[0]
    per = neg_ref.shape[2]
    acc = parts_ref[0]
    for p in range(1, nparts):
        acc = acc + parts_ref[p]
    s = acc * dinv_ref[0] + b_ref[0]
    for g in range(num_graphs):
        sg = s[:, g * per:(g + 1) * per]
        neg_ref[g] = -sg
        tanh_ref[g] = jnp.tanh(sg)


def _gather_body(perm_ref, ts_ref, feat_ref, out_ref):
    base = pl.program_id(0) * _GM
    rows = []
    for mi in range(_GM):
        r = perm_ref[base + mi]
        rows.append(feat_ref[r, 0] * ts_ref[r])
    for mi in range(_GM):
        out_ref[mi] = rows[mi]


def kernel(adj, feature, weight, bias):
    n, f = feature.shape
    num_graphs = 8
    per_graph = n // num_graphs
    k = int(math.ceil(0.5 * per_graph))
    wt = weight.reshape(1, f)

    nstrip = n // _TN
    dinv, parts = pl.pallas_call(
        _fused_body,
        out_shape=(jax.ShapeDtypeStruct((1, 1, n), jnp.float32),
                   jax.ShapeDtypeStruct((nstrip, 1, n), jnp.float32)),
        grid=(nstrip,),
        in_specs=[
            pl.BlockSpec((1, f), lambda i: (0, 0)),
            pl.BlockSpec((_TN, f), lambda i: (i, 0)),
            pl.BlockSpec((_TN, n), lambda i: (i, 0)),
        ],
        out_specs=(
            pl.BlockSpec((1, 1, _TN), lambda i: (0, 0, i)),
            pl.BlockSpec((1, 1, n), lambda i: (i, 0, 0)),
        ),
        compiler_params=pltpu.CompilerParams(
            dimension_semantics=("parallel",),
            vmem_limit_bytes=48 * 1024 * 1024),
    )(wt, feature, adj)

    neg3, tanh3 = pl.pallas_call(
        _combine_body,
        out_shape=(jax.ShapeDtypeStruct((num_graphs, 1, per_graph),
                                        jnp.float32),
                   jax.ShapeDtypeStruct((num_graphs, 1, per_graph),
                                        jnp.float32)),
        in_specs=[
            pl.BlockSpec(memory_space=pltpu.MemorySpace.SMEM),
            pl.BlockSpec((nstrip, 1, n), lambda: (0, 0, 0)),
            pl.BlockSpec((1, 1, n), lambda: (0, 0, 0)),
        ],
        out_specs=(
            pl.BlockSpec((num_graphs, 1, per_graph), lambda: (0, 0, 0)),
            pl.BlockSpec((num_graphs, 1, per_graph), lambda: (0, 0, 0)),
        ),
    )(bias, parts, dinv)

    order = jnp.argsort(neg3.reshape(num_graphs, per_graph), axis=1)
    offs = (jnp.arange(num_graphs, dtype=jnp.int32) * per_graph)[:, None]
    perm = (order[:, :k].astype(jnp.int32) + offs).reshape(-1)

    pooled = pl.pallas_call(
        _gather_body,
        out_shape=jax.ShapeDtypeStruct((perm.shape[0], f), jnp.float32),
        grid_spec=pltpu.PrefetchScalarGridSpec(
            num_scalar_prefetch=2,
            grid=(perm.shape[0] // _GM,),
            in_specs=[
                pl.BlockSpec((n, 1, f), lambda i, perm, ts: (0, 0, 0)),
            ],
            out_specs=pl.BlockSpec((_GM, f), lambda i, perm, ts: (i, 0)),
        ),
        compiler_params=pltpu.CompilerParams(
            dimension_semantics=("parallel",)),
    )(perm, tanh3.reshape(-1), feature.reshape(n, 1, f))

    next_batch_num_nodes = jnp.full((num_graphs,), k, dtype=jnp.int32)
    return pooled, perm, next_batch_num_nodes
